# Initial kernel scaffold; baseline (speedup 1.0000x reference)
import jax
import jax.numpy as jnp
from jax.experimental import pallas as pl


def kernel(text, emb_table, fc_w, fc_b):
    raise NotImplementedError("write your pallas kernel here")


# speedup vs baseline: 2.6539x; 2.6539x over previous
import functools

import jax
import jax.numpy as jnp
from jax import lax
from jax.experimental import pallas as pl
from jax.experimental.pallas import tpu as pltpu
from jax.experimental.pallas import tpu_sc as plsc

D = 128
L = 200
PCOLS = 16
NC, NS = 2, 16
NW = NC * NS
HALF = L // 2
CHUNK = 4


def _project_body(t_ref, w_ref, o_ref):
    o_ref[...] = jnp.dot(t_ref[...], w_ref[...],
                         preferred_element_type=jnp.float32)


def _project(table, w16):
    tm = 4000
    v = table.shape[0]
    return pl.pallas_call(
        _project_body,
        grid=(v // tm,),
        in_specs=[
            pl.BlockSpec((tm, D), lambda i: (i, 0)),
            pl.BlockSpec((D, PCOLS), lambda i: (0, 0)),
        ],
        out_specs=pl.BlockSpec((tm, PCOLS), lambda i: (i, 0)),
        out_shape=jax.ShapeDtypeStruct((v, PCOLS), jnp.float32),
        compiler_params=pltpu.CompilerParams(
            dimension_semantics=("arbitrary",)),
    )(table, w16)


def _make_pool(batch):
    rows_per_w = batch // NW
    n_chunks = rows_per_w // CHUNK

    @functools.partial(
        pl.kernel,
        out_type=jax.ShapeDtypeStruct((batch, PCOLS), jnp.float32),
        mesh=plsc.VectorSubcoreMesh(core_axis_name="c", subcore_axis_name="s",
                                    num_cores=NC, num_subcores=NS),
        scratch_types=[
            pltpu.VMEM((CHUNK, 2, HALF), jnp.int32),
            pltpu.VMEM((CHUNK, 2, HALF, PCOLS), jnp.float32),
            pltpu.VMEM((CHUNK, PCOLS), jnp.float32),
            pltpu.VMEM((PCOLS,), jnp.float32),
            pltpu.SemaphoreType.DMA,
        ],
        compiler_params=pltpu.CompilerParams(use_tc_tiling_on_sc=False),
    )
    def pool(text_hbm, proj_hbm, bias_hbm, out_hbm,
             idx_v, rows_v, out_v, bias_v, sem):
        wid = lax.axis_index("s") * NC + lax.axis_index("c")
        base0 = wid * rows_per_w
        pltpu.sync_copy(bias_hbm, bias_v)
        bias_vec = bias_v[...]

        def chunk_body(ci, carry):
            base = base0 + ci * CHUNK
            pltpu.sync_copy(text_hbm.at[pl.ds(base, CHUNK)], idx_v)
            cps = []
            for i in range(CHUNK):
                for j in range(2):
                    cps.append(pltpu.async_copy(
                        proj_hbm.at[idx_v.at[i, j]], rows_v.at[i, j], sem))
            for c in cps:
                c.wait()
            for i in range(CHUNK):
                accs = [rows_v[i, 0, t] for t in range(4)]
                for t in range(4, L):
                    accs[t % 4] = accs[t % 4] + rows_v[i, t // HALF, t % HALF]
                out_v[i] = (accs[0] + accs[1]) + (accs[2] + accs[3]) + bias_vec
            pltpu.sync_copy(out_v, out_hbm.at[pl.ds(base, CHUNK)])
            return carry

        lax.fori_loop(0, n_chunks, chunk_body, 0)

    return pool


def kernel(text, emb_table, fc_w, fc_b):
    batch = text.shape[0]
    ncls = fc_w.shape[1]
    idx = text.astype(jnp.int32).reshape(batch, 2, HALF)
    w16 = jnp.pad(fc_w, ((0, 0), (0, PCOLS - ncls))) * (1.0 / L)
    bias16 = jnp.pad(fc_b, (0, PCOLS - ncls))
    proj = _project(emb_table, w16)
    out16 = _make_pool(batch)(idx, proj, bias16)
    return out16[:, :ncls]


# speedup vs baseline: 7.2800x; 2.7432x over previous
import functools

import jax
import jax.numpy as jnp
from jax import lax
from jax.experimental import pallas as pl
from jax.experimental.pallas import tpu as pltpu
from jax.experimental.pallas import tpu_sc as plsc

D = 128
L = 200
PCOLS = 16
PACK = D // PCOLS
NC, NS = 2, 16
NW = NC * NS
G = 16
TROWS = G * L // 128
OROWS = G * PCOLS // 128


def _project_body(t_ref, w_ref, b_ref, o_ref):
    parts = [
        jnp.dot(t_ref[j], w_ref[...], preferred_element_type=jnp.float32)
        for j in range(PACK)
    ]
    o_ref[...] = jnp.concatenate(parts, axis=1) + b_ref[...]


def _project(table_g, w16, brow):
    tm = 1000
    vg = table_g.shape[1]
    return pl.pallas_call(
        _project_body,
        grid=(vg // tm,),
        in_specs=[
            pl.BlockSpec((PACK, tm, D), lambda i: (0, i, 0)),
            pl.BlockSpec((D, PCOLS), lambda i: (0, 0)),
            pl.BlockSpec((1, D), lambda i: (0, 0)),
        ],
        out_specs=pl.BlockSpec((tm, D), lambda i: (i, 0)),
        out_shape=jax.ShapeDtypeStruct((vg, D), jnp.float32),
        compiler_params=pltpu.CompilerParams(
            dimension_semantics=("arbitrary",)),
    )(table_g, w16, brow)


def _make_pool(batch):
    rows_per_w = batch // NW
    n_chunks = rows_per_w // G
    t_per_w = rows_per_w * L // 128
    o_per_w = rows_per_w * PCOLS // 128

    @functools.partial(
        pl.kernel,
        out_type=jax.ShapeDtypeStruct((batch * PCOLS // 128, 128),
                                      jnp.float32),
        mesh=plsc.VectorSubcoreMesh(core_axis_name="c", subcore_axis_name="s",
                                    num_cores=NC, num_subcores=NS),
        scratch_types=[
            pltpu.VMEM((2, TROWS, 128), jnp.int32),
            pltpu.VMEM((2, TROWS * 128, PCOLS), jnp.float32),
            pltpu.VMEM((2, OROWS, 128), jnp.float32),
            pltpu.SemaphoreType.DMA,
            pltpu.SemaphoreType.DMA,
            pltpu.SemaphoreType.DMA,
            pltpu.SemaphoreType.DMA,
            pltpu.SemaphoreType.DMA,
            pltpu.SemaphoreType.DMA,
        ],
        compiler_params=pltpu.CompilerParams(use_tc_tiling_on_sc=False),
    )
    def pool(text_hbm, proj_hbm, out_hbm,
             idx_v, rows_v, out_v, gsem0, gsem1, isem0, isem1, osem0, osem1):
        gsems = (gsem0, gsem1)
        isems = (isem0, isem1)
        osems = (osem0, osem1)
        wid = lax.axis_index("s") * NC + lax.axis_index("c")
        tbase = wid * t_per_w
        obase = wid * o_per_w

        def issue_idx(ci, slot):
            pltpu.async_copy(text_hbm.at[pl.ds(tbase + ci * TROWS, TROWS)],
                             idx_v.at[slot], isems[slot])

        def wait_idx(ci, slot):
            pltpu.make_async_copy(
                text_hbm.at[pl.ds(tbase + ci * TROWS, TROWS)],
                idx_v.at[slot], isems[slot]).wait()

        def issue_gathers(slot):
            for r in range(TROWS):
                pltpu.async_copy(proj_hbm.at[idx_v.at[slot, r]],
                                 rows_v.at[slot, pl.ds(r * 128, 128)],
                                 gsems[slot])

        def wait_gathers(slot):
            for r in range(TROWS):
                pltpu.make_async_copy(proj_hbm.at[idx_v.at[slot, r]],
                                      rows_v.at[slot, pl.ds(r * 128, 128)],
                                      gsems[slot]).wait()

        def issue_out(ci, slot):
            pltpu.async_copy(out_v.at[slot],
                             out_hbm.at[pl.ds(obase + ci * OROWS, OROWS)],
                             osems[slot])

        def drain_out(slot):
            pltpu.make_async_copy(out_v.at[slot],
                                  out_hbm.at[pl.ds(obase, OROWS)],
                                  osems[slot]).wait()

        def compute(ci, slot):
            def row_body(bi, carry):
                f0 = bi * L
                accs = [rows_v[slot, f0 + t] for t in range(4)]
                for t in range(4, L):
                    accs[t % 4] = accs[t % 4] + rows_v[slot, f0 + t]
                acc = (accs[0] + accs[1]) + (accs[2] + accs[3])
                out_v[slot, bi // (128 // PCOLS),
                      pl.ds((bi % (128 // PCOLS)) * PCOLS, PCOLS)] = acc
                return carry

            lax.fori_loop(0, G, row_body, 0)
            issue_out(ci, slot)

        pltpu.sync_copy(text_hbm.at[pl.ds(tbase, TROWS)], idx_v.at[0])
        issue_gathers(0)
        issue_idx(1, 1)

        def body(k, carry):
            ci0 = 2 * k
            for s in range(2):
                ci = ci0 + s
                wait_gathers(s)
                wait_idx(ci + 1, 1 - s)
                issue_gathers(1 - s)
                issue_idx(ci + 2, s)
                pl.when(k >= 1)(functools.partial(drain_out, s))
                compute(ci, s)
            return carry

        lax.fori_loop(0, (n_chunks - 2) // 2, body, 0)

        ci = n_chunks - 2
        wait_gathers(0)
        wait_idx(ci + 1, 1)
        issue_gathers(1)
        drain_out(0)
        compute(ci, 0)
        wait_gathers(1)
        drain_out(1)
        compute(ci + 1, 1)
        drain_out(0)
        drain_out(1)

    return pool


def kernel(text, emb_table, fc_w, fc_b):
    batch = text.shape[0]
    vocab = emb_table.shape[0]
    vg = vocab // PACK
    ncls = fc_w.shape[1]
    t32 = text.astype(jnp.int32)
    r32 = (t32 % vg) * PACK + t32 // vg
    idx128 = r32.reshape(batch * L // 128, 128)
    w16 = jnp.pad(fc_w, ((0, 0), (0, PCOLS - ncls))) * (1.0 / L)
    brow = jnp.tile(jnp.pad(fc_b, (0, PCOLS - ncls)) * (1.0 / L),
                    PACK)[None, :]
    table_g = emb_table.reshape(PACK, vg, D)
    proj = _project(table_g, w16, brow).reshape(vocab, PCOLS)
    out = _make_pool(batch)(idx128, proj)
    return out.reshape(batch, PCOLS)[:, :ncls]


# speedup vs baseline: 7.5890x; 1.0425x over previous
import functools

import jax
import jax.numpy as jnp
from jax import lax
from jax.experimental import pallas as pl
from jax.experimental.pallas import tpu as pltpu
from jax.experimental.pallas import tpu_sc as plsc

D = 128
L = 200
PCOLS = 16
PACK = D // PCOLS
NC, NS = 2, 16
NW = NC * NS
G = 16
TROWS = G * L // 128
OROWS = G * PCOLS // 128


def _project_body(t_ref, w_ref, b_ref, o_ref):
    parts = [
        jnp.dot(t_ref[j], w_ref[...], preferred_element_type=jnp.float32)
        for j in range(PACK)
    ]
    o_ref[...] = jnp.concatenate(parts, axis=1) + b_ref[...]


def _project(table_g, w16, brow):
    tm = 1000
    vg = table_g.shape[1]
    return pl.pallas_call(
        _project_body,
        grid=(vg // tm,),
        in_specs=[
            pl.BlockSpec((PACK, tm, D), lambda i: (0, i, 0)),
            pl.BlockSpec((D, PCOLS), lambda i: (0, 0)),
            pl.BlockSpec((1, D), lambda i: (0, 0)),
        ],
        out_specs=pl.BlockSpec((tm, D), lambda i: (i, 0)),
        out_shape=jax.ShapeDtypeStruct((vg, D), jnp.float32),
        compiler_params=pltpu.CompilerParams(
            dimension_semantics=("arbitrary",)),
    )(table_g, w16, brow)


NL = 4
NBLK = L // NL
Q = 4


def _make_pool(batch):
    rows_per_w = batch // NW
    o_per_w = rows_per_w * PCOLS // 128

    @functools.partial(
        pl.kernel,
        out_type=jax.ShapeDtypeStruct((batch * PCOLS // 128, 128),
                                      jnp.float32),
        mesh=plsc.VectorSubcoreMesh(core_axis_name="c", subcore_axis_name="s",
                                    num_cores=NC, num_subcores=NS),
        scratch_types=[
            pltpu.VMEM((2, Q, 2 * NL, 128), jnp.int32),
            pltpu.VMEM((2, NL * rows_per_w, PCOLS), jnp.float32),
            pltpu.VMEM((Q, 128), jnp.int32),
            pltpu.VMEM((128, PCOLS), jnp.float32),
            pltpu.VMEM((rows_per_w, PCOLS), jnp.float32),
            pltpu.VMEM((64, 128), jnp.float32),
            pltpu.VMEM_SHARED((NS * rows_per_w, PCOLS), jnp.float32),
            pltpu.SemaphoreType.DMA,
            pltpu.SemaphoreType.DMA,
            pltpu.SemaphoreType.DMA,
            pltpu.SemaphoreType.DMA,
            pltpu.SemaphoreType.DMA,
            pltpu.SemaphoreType.DMA,
            pltpu.SemaphoreType.DMA,
        ],
        compiler_params=pltpu.CompilerParams(use_tc_tiling_on_sc=False),
    )
    def pool(text_hbm, proj_hbm, out_hbm,
             idx_v, rows_v, ramp_v, zero_v, stage_v, ostage_v, acc_sh,
             gsem0, gsem1, isem0, isem1, ssem0, ssem1, zsem):
        gsems = (gsem0, gsem1)
        isems = (isem0, isem1)
        ssems = (ssem0, ssem1)
        sid = lax.axis_index("s")
        wid = sid * NC + lax.axis_index("c")
        cbase = wid * rows_per_w
        abase = sid * rows_per_w
        obase = wid * o_per_w

        cb4 = wid * Q

        def issue_idx(f, slot):
            for q in range(Q):
                pltpu.async_copy(text_hbm.at[f, cb4 + q],
                                 idx_v.at[slot, q], isems[slot])

        def wait_idx(f, slot):
            for q in range(Q):
                pltpu.make_async_copy(text_hbm.at[f, cb4 + q],
                                      idx_v.at[slot, q], isems[slot]).wait()

        def issue_gathers(slot, fslot, loff):
            for q in range(Q):
                for li in range(NL):
                    pltpu.async_copy(
                        proj_hbm.at[idx_v.at[fslot, q, loff + li]],
                        rows_v.at[slot, pl.ds((li * Q + q) * 128, 128)],
                        gsems[slot])

        def wait_gathers(slot, fslot, loff):
            for q in range(Q):
                for li in range(NL):
                    pltpu.make_async_copy(
                        proj_hbm.at[idx_v.at[fslot, q, loff + li]],
                        rows_v.at[slot, pl.ds((li * Q + q) * 128, 128)],
                        gsems[slot]).wait()

        def issue_scatters(slot):
            for q in range(Q):
                for li in range(NL):
                    pltpu.async_copy(
                        rows_v.at[slot, pl.ds((li * Q + q) * 128, 128)],
                        acc_sh.at[ramp_v.at[q]], ssems[slot], add=True)

        def drain_scatters(slot):
            for q in range(Q):
                for li in range(NL):
                    pltpu.make_async_copy(
                        rows_v.at[slot, pl.ds((li * Q + q) * 128, 128)],
                        acc_sh.at[ramp_v.at[q]], ssems[slot]).wait()

        iota16 = lax.iota(jnp.int32, 16)
        for q in range(Q):
            for kk in range(8):
                ramp_v[q, pl.ds(kk * 16, 16)] = (
                    iota16 + (abase + q * 128 + kk * 16))

        def zrow(i, carry):
            zero_v[i] = jnp.zeros((PCOLS,), jnp.float32)
            return carry

        lax.fori_loop(0, 128, zrow, 0)
        for q in range(Q):
            pltpu.async_copy(zero_v,
                             acc_sh.at[pl.ds(abase + q * 128, 128)], zsem)
        for q in range(Q):
            pltpu.make_async_copy(zero_v,
                                  acc_sh.at[pl.ds(abase, 128)], zsem).wait()

        issue_idx(0, 0)
        wait_idx(0, 0)
        issue_gathers(0, 0, 0)

        def body(k, carry):
            f2 = 2 * k
            wait_gathers(0, 0, 0)
            pl.when(k >= 1)(functools.partial(drain_scatters, 1))
            issue_idx(f2 + 1, 1)
            issue_gathers(1, 0, NL)
            issue_scatters(0)
            wait_gathers(1, 0, NL)
            drain_scatters(0)
            wait_idx(f2 + 1, 1)
            issue_gathers(0, 1, 0)
            issue_scatters(1)
            wait_gathers(0, 1, 0)
            drain_scatters(1)
            issue_idx(f2 + 2, 0)
            issue_gathers(1, 1, NL)
            issue_scatters(0)
            wait_gathers(1, 1, NL)
            drain_scatters(0)
            wait_idx(f2 + 2, 0)
            issue_gathers(0, 0, 0)
            issue_scatters(1)
            return carry

        lax.fori_loop(0, (NBLK - 2) // 4, body, 0)

        wait_gathers(0, 0, 0)
        drain_scatters(1)
        issue_gathers(1, 0, NL)
        issue_scatters(0)
        wait_gathers(1, 0, NL)
        drain_scatters(0)
        issue_scatters(1)
        drain_scatters(1)

        pltpu.sync_copy(acc_sh.at[pl.ds(abase, rows_per_w)], stage_v)

        def orow(g, carry):
            for j in range(8):
                ostage_v[g, pl.ds(j * PCOLS, PCOLS)] = stage_v[g * 8 + j]
            return carry

        lax.fori_loop(0, o_per_w, orow, 0)
        pltpu.sync_copy(ostage_v, out_hbm.at[pl.ds(obase, o_per_w)])

    return pool


def kernel(text, emb_table, fc_w, fc_b):
    batch = text.shape[0]
    vocab = emb_table.shape[0]
    vg = vocab // PACK
    ncls = fc_w.shape[1]
    t32 = text.astype(jnp.int32)
    r32 = (t32 % vg) * PACK + t32 // vg
    text4 = r32.reshape(batch // 128, 128, L // 8, 8).transpose(2, 0, 3, 1)
    w16 = jnp.pad(fc_w, ((0, 0), (0, PCOLS - ncls))) * (1.0 / L)
    brow = jnp.tile(jnp.pad(fc_b, (0, PCOLS - ncls)) * (1.0 / L),
                    PACK)[None, :]
    table_g = emb_table.reshape(PACK, vg, D)
    proj = _project(table_g, w16, brow).reshape(vocab, PCOLS)
    out = _make_pool(batch)(text4, proj)
    return out.reshape(batch, PCOLS)[:, :ncls]


# speedup vs baseline: 7.6282x; 1.0052x over previous
import functools

import jax
import jax.numpy as jnp
from jax import lax
from jax.experimental import pallas as pl
from jax.experimental.pallas import tpu as pltpu
from jax.experimental.pallas import tpu_sc as plsc

D = 128
L = 200
PCOLS = 16
PACK = D // PCOLS
NC, NS = 2, 16
NW = NC * NS
G = 16
TROWS = G * L // 128
OROWS = G * PCOLS // 128


def _project_body(t_ref, w_ref, b_ref, o_ref):
    parts = [
        jnp.dot(t_ref[j], w_ref[...], preferred_element_type=jnp.float32)
        for j in range(PACK)
    ]
    o_ref[...] = jnp.concatenate(parts, axis=1) + b_ref[...]


def _project(table_g, w16, brow):
    tm = 1000
    vg = table_g.shape[1]
    return pl.pallas_call(
        _project_body,
        grid=(vg // tm,),
        in_specs=[
            pl.BlockSpec((PACK, tm, D), lambda i: (0, i, 0)),
            pl.BlockSpec((D, PCOLS), lambda i: (0, 0)),
            pl.BlockSpec((1, D), lambda i: (0, 0)),
        ],
        out_specs=pl.BlockSpec((tm, D), lambda i: (i, 0)),
        out_shape=jax.ShapeDtypeStruct((vg, D), jnp.float32),
        compiler_params=pltpu.CompilerParams(
            dimension_semantics=("arbitrary",)),
    )(table_g, w16, brow)


NL = 4
NBLK = L // NL
Q = 4


def _make_pool(batch):
    rows_per_w = batch // NW
    o_per_w = rows_per_w * PCOLS // 128

    @functools.partial(
        pl.kernel,
        out_type=jax.ShapeDtypeStruct((batch * PCOLS // 128, 128),
                                      jnp.float32),
        mesh=plsc.VectorSubcoreMesh(core_axis_name="c", subcore_axis_name="s",
                                    num_cores=NC, num_subcores=NS),
        scratch_types=[
            pltpu.VMEM((2, Q, 2 * NL, 128), jnp.int32),
            pltpu.VMEM((2, NL * rows_per_w, PCOLS), jnp.float32),
            pltpu.VMEM((rows_per_w, PCOLS), jnp.float32),
            pltpu.VMEM((64, 128), jnp.float32),
            pltpu.SemaphoreType.DMA,
            pltpu.SemaphoreType.DMA,
            pltpu.SemaphoreType.DMA,
            pltpu.SemaphoreType.DMA,
        ],
        compiler_params=pltpu.CompilerParams(use_tc_tiling_on_sc=False),
    )
    def pool(text_hbm, proj_hbm, out_hbm,
             idx_v, rows_v, acc_v, ostage_v,
             gsem0, gsem1, isem0, isem1):
        gsems = (gsem0, gsem1)
        isems = (isem0, isem1)
        sid = lax.axis_index("s")
        wid = sid * NC + lax.axis_index("c")
        obase = wid * o_per_w

        cb4 = wid * Q

        def issue_idx(f, slot):
            for q in range(Q):
                pltpu.async_copy(text_hbm.at[f, cb4 + q],
                                 idx_v.at[slot, q], isems[slot])

        def wait_idx(f, slot):
            for q in range(Q):
                pltpu.make_async_copy(text_hbm.at[f, cb4 + q],
                                      idx_v.at[slot, q], isems[slot]).wait()

        def issue_gathers(slot, fslot, loff):
            for q in range(Q):
                for li in range(NL):
                    pltpu.async_copy(
                        proj_hbm.at[idx_v.at[fslot, q, loff + li]],
                        rows_v.at[slot, pl.ds((li * Q + q) * 128, 128)],
                        gsems[slot])

        def wait_gathers(slot, fslot, loff):
            for q in range(Q):
                for li in range(NL):
                    pltpu.make_async_copy(
                        proj_hbm.at[idx_v.at[fslot, q, loff + li]],
                        rows_v.at[slot, pl.ds((li * Q + q) * 128, 128)],
                        gsems[slot]).wait()

        def accumulate(slot):
            def jbody(j, carry):
                for q in range(Q):
                    base = q * 128 + j
                    v = rows_v[slot, base]
                    for li in range(1, NL):
                        v = v + rows_v[slot, li * Q * 128 + base]
                    plsc.addupdate(acc_v.at[base], v)
                return carry

            lax.fori_loop(0, 128, jbody, 0)

        def zrow(i, carry):
            acc_v[i] = jnp.zeros((PCOLS,), jnp.float32)
            return carry

        lax.fori_loop(0, rows_per_w, zrow, 0)

        issue_idx(0, 0)
        wait_idx(0, 0)
        issue_gathers(0, 0, 0)

        def body(k, carry):
            f2 = 2 * k
            wait_gathers(0, 0, 0)
            issue_idx(f2 + 1, 1)
            issue_gathers(1, 0, NL)
            accumulate(0)
            wait_gathers(1, 0, NL)
            wait_idx(f2 + 1, 1)
            issue_gathers(0, 1, 0)
            accumulate(1)
            wait_gathers(0, 1, 0)
            issue_idx(f2 + 2, 0)
            issue_gathers(1, 1, NL)
            accumulate(0)
            wait_gathers(1, 1, NL)
            wait_idx(f2 + 2, 0)
            issue_gathers(0, 0, 0)
            accumulate(1)
            return carry

        lax.fori_loop(0, (NBLK - 2) // 4, body, 0)

        wait_gathers(0, 0, 0)
        issue_gathers(1, 0, NL)
        accumulate(0)
        wait_gathers(1, 0, NL)
        accumulate(1)

        def orow(g, carry):
            for j in range(8):
                ostage_v[g, pl.ds(j * PCOLS, PCOLS)] = acc_v[g * 8 + j]
            return carry

        lax.fori_loop(0, o_per_w, orow, 0)
        pltpu.sync_copy(ostage_v, out_hbm.at[pl.ds(obase, o_per_w)])

    return pool


def kernel(text, emb_table, fc_w, fc_b):
    batch = text.shape[0]
    vocab = emb_table.shape[0]
    vg = vocab // PACK
    ncls = fc_w.shape[1]
    t32 = text.astype(jnp.int32)
    r32 = (t32 % vg) * PACK + t32 // vg
    text4 = r32.reshape(batch // 128, 128, L // 8, 8).transpose(2, 0, 3, 1)
    w16 = jnp.pad(fc_w, ((0, 0), (0, PCOLS - ncls))) * (1.0 / L)
    brow = jnp.tile(jnp.pad(fc_b, (0, PCOLS - ncls)) * (1.0 / L),
                    PACK)[None, :]
    table_g = emb_table.reshape(PACK, vg, D)
    proj = _project(table_g, w16, brow).reshape(vocab, PCOLS)
    out = _make_pool(batch)(text4, proj)
    return out.reshape(batch, PCOLS)[:, :ncls]


# speedup vs baseline: 7.6930x; 1.0085x over previous
import functools

import jax
import jax.numpy as jnp
from jax import lax
from jax.experimental import pallas as pl
from jax.experimental.pallas import tpu as pltpu
from jax.experimental.pallas import tpu_sc as plsc

D = 128
L = 200
PCOLS = 16
PACK = D // PCOLS
NC, NS = 2, 16
NW = NC * NS
G = 16
TROWS = G * L // 128
OROWS = G * PCOLS // 128


def _project_body(t_ref, w_ref, b_ref, o_ref):
    acc = b_ref[...] + jnp.dot(t_ref[0], w_ref[0],
                               preferred_element_type=jnp.float32)
    for j in range(1, PACK):
        acc = acc + jnp.dot(t_ref[j], w_ref[j],
                            preferred_element_type=jnp.float32)
    o_ref[...] = acc


def _project(table_g, wbd, brow):
    tm = 1000
    vg = table_g.shape[1]
    return pl.pallas_call(
        _project_body,
        grid=(vg // tm,),
        in_specs=[
            pl.BlockSpec((PACK, tm, D), lambda i: (0, i, 0)),
            pl.BlockSpec((PACK, D, D), lambda i: (0, 0, 0)),
            pl.BlockSpec((1, D), lambda i: (0, 0)),
        ],
        out_specs=pl.BlockSpec((tm, D), lambda i: (i, 0)),
        out_shape=jax.ShapeDtypeStruct((vg, D), jnp.float32),
        compiler_params=pltpu.CompilerParams(
            dimension_semantics=("arbitrary",)),
    )(table_g, wbd, brow)


NL = 4
NBLK = L // NL
Q = 4


def _make_pool(batch):
    rows_per_w = batch // NW
    o_per_w = rows_per_w * PCOLS // 128

    @functools.partial(
        pl.kernel,
        out_type=jax.ShapeDtypeStruct((batch * PCOLS // 128, 128),
                                      jnp.float32),
        mesh=plsc.VectorSubcoreMesh(core_axis_name="c", subcore_axis_name="s",
                                    num_cores=NC, num_subcores=NS),
        scratch_types=[
            pltpu.VMEM((2, Q, 2 * NL, 128), jnp.int32),
            pltpu.VMEM((2, NL * rows_per_w, PCOLS), jnp.float32),
            pltpu.VMEM((rows_per_w, PCOLS), jnp.float32),
            pltpu.VMEM((64, 128), jnp.float32),
            pltpu.SemaphoreType.DMA,
            pltpu.SemaphoreType.DMA,
            pltpu.SemaphoreType.DMA,
            pltpu.SemaphoreType.DMA,
        ],
        compiler_params=pltpu.CompilerParams(use_tc_tiling_on_sc=False),
    )
    def pool(text_hbm, proj_hbm, out_hbm,
             idx_v, rows_v, acc_v, ostage_v,
             gsem0, gsem1, isem0, isem1):
        gsems = (gsem0, gsem1)
        isems = (isem0, isem1)
        sid = lax.axis_index("s")
        wid = sid * NC + lax.axis_index("c")
        obase = wid * o_per_w

        cb4 = wid * Q

        def issue_idx(f, slot):
            for q in range(Q):
                pltpu.async_copy(text_hbm.at[f, cb4 + q],
                                 idx_v.at[slot, q], isems[slot])

        def wait_idx(f, slot):
            for q in range(Q):
                pltpu.make_async_copy(text_hbm.at[f, cb4 + q],
                                      idx_v.at[slot, q], isems[slot]).wait()

        def issue_gathers(slot, fslot, loff):
            for q in range(Q):
                for li in range(NL):
                    pltpu.async_copy(
                        proj_hbm.at[idx_v.at[fslot, q, loff + li]],
                        rows_v.at[slot, pl.ds((li * Q + q) * 128, 128)],
                        gsems[slot])

        def wait_gathers(slot, fslot, loff):
            for q in range(Q):
                for li in range(NL):
                    pltpu.make_async_copy(
                        proj_hbm.at[idx_v.at[fslot, q, loff + li]],
                        rows_v.at[slot, pl.ds((li * Q + q) * 128, 128)],
                        gsems[slot]).wait()

        def accumulate(slot):
            def jbody(j, carry):
                for q in range(Q):
                    base = q * 128 + j
                    v = rows_v[slot, base]
                    for li in range(1, NL):
                        v = v + rows_v[slot, li * Q * 128 + base]
                    plsc.addupdate(acc_v.at[base], v)
                return carry

            lax.fori_loop(0, 128, jbody, 0)

        def zrow(i, carry):
            acc_v[i] = jnp.zeros((PCOLS,), jnp.float32)
            return carry

        lax.fori_loop(0, rows_per_w, zrow, 0)

        issue_idx(0, 0)
        wait_idx(0, 0)
        issue_gathers(0, 0, 0)

        def body(k, carry):
            f2 = 2 * k
            wait_gathers(0, 0, 0)
            issue_idx(f2 + 1, 1)
            issue_gathers(1, 0, NL)
            accumulate(0)
            wait_gathers(1, 0, NL)
            wait_idx(f2 + 1, 1)
            issue_gathers(0, 1, 0)
            accumulate(1)
            wait_gathers(0, 1, 0)
            issue_idx(f2 + 2, 0)
            issue_gathers(1, 1, NL)
            accumulate(0)
            wait_gathers(1, 1, NL)
            wait_idx(f2 + 2, 0)
            issue_gathers(0, 0, 0)
            accumulate(1)
            return carry

        lax.fori_loop(0, (NBLK - 2) // 4, body, 0)

        wait_gathers(0, 0, 0)
        issue_gathers(1, 0, NL)
        accumulate(0)
        wait_gathers(1, 0, NL)
        accumulate(1)

        def orow(g, carry):
            for j in range(8):
                ostage_v[g, pl.ds(j * PCOLS, PCOLS)] = acc_v[g * 8 + j]
            return carry

        lax.fori_loop(0, o_per_w, orow, 0)
        pltpu.sync_copy(ostage_v, out_hbm.at[pl.ds(obase, o_per_w)])

    return pool


def kernel(text, emb_table, fc_w, fc_b):
    batch = text.shape[0]
    vocab = emb_table.shape[0]
    vg = vocab // PACK
    ncls = fc_w.shape[1]
    t32 = text.astype(jnp.int32)
    r32 = (t32 % vg) * PACK + t32 // vg
    text4 = r32.reshape(batch // 128, 128, L // 8, 8).transpose(2, 0, 3, 1)
    w16 = jnp.pad(fc_w, ((0, 0), (0, PCOLS - ncls))) * (1.0 / L)
    wbd = jnp.kron(jnp.eye(PACK, dtype=jnp.float32), w16).reshape(PACK, D, D)
    brow = jnp.tile(jnp.pad(fc_b, (0, PCOLS - ncls)) * (1.0 / L),
                    PACK)[None, :]
    table_g = emb_table.reshape(PACK, vg, D)
    proj = _project(table_g, wbd, brow).reshape(vocab, PCOLS)
    out = _make_pool(batch)(text4, proj)
    return out.reshape(batch, PCOLS)[:, :ncls]


# speedup vs baseline: 8.6948x; 1.1302x over previous
import functools

import jax
import jax.numpy as jnp
from jax import lax
from jax.experimental import pallas as pl
from jax.experimental.pallas import tpu as pltpu
from jax.experimental.pallas import tpu_sc as plsc

D = 128
L = 200
PCOLS = 16
PACK = D // PCOLS
NC, NS = 2, 16
NW = NC * NS
G = 16
TROWS = G * L // 128
OROWS = G * PCOLS // 128


def _project_body(t_ref, w_ref, b_ref, o_ref):
    acc = b_ref[...] + jnp.dot(t_ref[0], w_ref[0],
                               preferred_element_type=jnp.float32)
    for j in range(1, PACK):
        acc = acc + jnp.dot(t_ref[j], w_ref[j],
                            preferred_element_type=jnp.float32)
    o_ref[...] = acc


def _project(table_g, wbd, brow):
    tm = 5000
    vg = table_g.shape[1]
    return pl.pallas_call(
        _project_body,
        grid=(vg // tm,),
        in_specs=[
            pl.BlockSpec((PACK, tm, D), lambda i: (0, i, 0)),
            pl.BlockSpec((PACK, D, D), lambda i: (0, 0, 0)),
            pl.BlockSpec((1, D), lambda i: (0, 0)),
        ],
        out_specs=pl.BlockSpec((tm, D), lambda i: (i, 0)),
        out_shape=jax.ShapeDtypeStruct((vg, D), jnp.float32),
        compiler_params=pltpu.CompilerParams(
            dimension_semantics=("arbitrary",)),
    )(table_g, wbd, brow)


NL = 4
NBLK = L // NL
Q = 4


def _make_pool(batch):
    rows_per_w = batch // NW
    o_per_w = rows_per_w * PCOLS // 128

    @functools.partial(
        pl.kernel,
        out_type=jax.ShapeDtypeStruct((batch * PCOLS // 128, 128),
                                      jnp.float32),
        mesh=plsc.VectorSubcoreMesh(core_axis_name="c", subcore_axis_name="s",
                                    num_cores=NC, num_subcores=NS),
        scratch_types=[
            pltpu.VMEM((2, Q, 2 * NL, 128), jnp.int32),
            pltpu.VMEM((2, NL * rows_per_w, PCOLS), jnp.float32),
            pltpu.VMEM((rows_per_w, PCOLS), jnp.float32),
            pltpu.VMEM((64, 128), jnp.float32),
            pltpu.SemaphoreType.DMA,
            pltpu.SemaphoreType.DMA,
            pltpu.SemaphoreType.DMA,
            pltpu.SemaphoreType.DMA,
        ],
        compiler_params=pltpu.CompilerParams(use_tc_tiling_on_sc=False),
    )
    def pool(text_hbm, proj_hbm, out_hbm,
             idx_v, rows_v, acc_v, ostage_v,
             gsem0, gsem1, isem0, isem1):
        gsems = (gsem0, gsem1)
        isems = (isem0, isem1)
        sid = lax.axis_index("s")
        wid = sid * NC + lax.axis_index("c")
        obase = wid * o_per_w

        cb4 = wid * Q

        def issue_idx(f, slot):
            for q in range(Q):
                pltpu.async_copy(text_hbm.at[f, cb4 + q],
                                 idx_v.at[slot, q], isems[slot])

        def wait_idx(f, slot):
            for q in range(Q):
                pltpu.make_async_copy(text_hbm.at[f, cb4 + q],
                                      idx_v.at[slot, q], isems[slot]).wait()

        def issue_gathers(slot, fslot, loff):
            for q in range(Q):
                for li in range(NL):
                    pltpu.async_copy(
                        proj_hbm.at[idx_v.at[fslot, q, loff + li]],
                        rows_v.at[slot, pl.ds((li * Q + q) * 128, 128)],
                        gsems[slot])

        def wait_gathers(slot, fslot, loff):
            for q in range(Q):
                for li in range(NL):
                    pltpu.make_async_copy(
                        proj_hbm.at[idx_v.at[fslot, q, loff + li]],
                        rows_v.at[slot, pl.ds((li * Q + q) * 128, 128)],
                        gsems[slot]).wait()

        def accumulate(slot):
            def jbody(j, carry):
                for q in range(Q):
                    base = q * 128 + j
                    v = rows_v[slot, base]
                    for li in range(1, NL):
                        v = v + rows_v[slot, li * Q * 128 + base]
                    plsc.addupdate(acc_v.at[base], v)
                return carry

            lax.fori_loop(0, 128, jbody, 0)

        def zrow(i, carry):
            acc_v[i] = jnp.zeros((PCOLS,), jnp.float32)
            return carry

        lax.fori_loop(0, rows_per_w, zrow, 0)

        issue_idx(0, 0)
        wait_idx(0, 0)
        issue_gathers(0, 0, 0)

        def body(k, carry):
            f2 = 2 * k
            wait_gathers(0, 0, 0)
            issue_idx(f2 + 1, 1)
            issue_gathers(1, 0, NL)
            accumulate(0)
            wait_gathers(1, 0, NL)
            wait_idx(f2 + 1, 1)
            issue_gathers(0, 1, 0)
            accumulate(1)
            wait_gathers(0, 1, 0)
            issue_idx(f2 + 2, 0)
            issue_gathers(1, 1, NL)
            accumulate(0)
            wait_gathers(1, 1, NL)
            wait_idx(f2 + 2, 0)
            issue_gathers(0, 0, 0)
            accumulate(1)
            return carry

        lax.fori_loop(0, (NBLK - 2) // 4, body, 0)

        wait_gathers(0, 0, 0)
        issue_gathers(1, 0, NL)
        accumulate(0)
        wait_gathers(1, 0, NL)
        accumulate(1)

        def orow(g, carry):
            for j in range(8):
                ostage_v[g, pl.ds(j * PCOLS, PCOLS)] = acc_v[g * 8 + j]
            return carry

        lax.fori_loop(0, o_per_w, orow, 0)
        pltpu.sync_copy(ostage_v, out_hbm.at[pl.ds(obase, o_per_w)])

    return pool


def kernel(text, emb_table, fc_w, fc_b):
    batch = text.shape[0]
    vocab = emb_table.shape[0]
    vg = vocab // PACK
    ncls = fc_w.shape[1]
    t32 = text.astype(jnp.int32)
    r32 = (t32 % vg) * PACK + t32 // vg
    text4 = r32.reshape(batch // 128, 128, L // 8, 8).transpose(2, 0, 3, 1)
    w16 = jnp.pad(fc_w, ((0, 0), (0, PCOLS - ncls))) * (1.0 / L)
    wbd = jnp.kron(jnp.eye(PACK, dtype=jnp.float32), w16).reshape(PACK, D, D)
    brow = jnp.tile(jnp.pad(fc_b, (0, PCOLS - ncls)) * (1.0 / L),
                    PACK)[None, :]
    table_g = emb_table.reshape(PACK, vg, D)
    proj = _project(table_g, wbd, brow).reshape(vocab, PCOLS)
    out = _make_pool(batch)(text4, proj)
    return out.reshape(batch, PCOLS)[:, :ncls]
